# Initial kernel scaffold; baseline (speedup 1.0000x reference)
import jax
import jax.numpy as jnp
from jax.experimental import pallas as pl


def kernel(X_HypGNet, X_CGNet, X_SGNet, edge_index_hyp, edge_index_cg, edge_index_sg, W_hyp, b_hyp, W_cg, b_cg, W_sg, b_sg):
    raise NotImplementedError("write your pallas kernel here")


# speedup vs baseline: 16.5804x; 16.5804x over previous
import jax
import jax.numpy as jnp
from jax import lax
from jax.experimental import pallas as pl
from jax.experimental.pallas import tpu as pltpu
from jax.experimental.pallas import tpu_sc as plsc

N = 10000
D = 128
E = 320000
NC = 2
NS = 16
NW = NC * NS
CHUNK = 128
EPW = E // NW
KW = -(-EPW // CHUNK)
EPW_PAD = KW * CHUNK
NPAD = 10112
RPS = NPAD // NS
DUMMY = N
BR = 1000


def _deg_body(d0, d1, d2, zz, o0, o1, o2, dst_v, ones_v, accum):
    c = lax.axis_index("c")
    s = lax.axis_index("s")
    w = c * NS + s
    for i in range(CHUNK):
        ones_v[i] = jnp.ones((16,), jnp.float32)
    for dstp, out in ((d0, o0), (d1, o1), (d2, o2)):
        pltpu.sync_copy(zz, accum.at[pl.ds(s * RPS, RPS)])
        plsc.subcore_barrier()
        pltpu.sync_copy(dstp.at[w], dst_v)

        def body(j, carry):
            pltpu.sync_copy(ones_v, accum.at[dst_v.at[j]], add=True)
            return carry

        lax.fori_loop(0, KW, body, 0)
        plsc.subcore_barrier()
        pltpu.sync_copy(accum.at[pl.ds(s * RPS, RPS)],
                        out.at[c, pl.ds(s * RPS, RPS)])


def _edge_body(y0, s0, d0, y1, s1, d1, y2, s2, d2, zz,
               p0, p1, p2, src_v, dst_v, rows_v, accum, sem):
    c = lax.axis_index("c")
    s = lax.axis_index("s")
    w = c * NS + s
    for y, srcp, dstp, part in ((y0, s0, d0, p0), (y1, s1, d1, p1),
                                (y2, s2, d2, p2)):
        pltpu.sync_copy(zz, accum.at[pl.ds(s * RPS, RPS)])
        plsc.subcore_barrier()
        pltpu.sync_copy(srcp.at[w], src_v)
        pltpu.sync_copy(dstp.at[w], dst_v)

        def body(j, carry):
            pltpu.async_copy(y.at[src_v.at[j]], rows_v, sem).wait()
            pltpu.sync_copy(rows_v, accum.at[dst_v.at[j]], add=True)
            return carry

        lax.fori_loop(0, KW, body, 0)
        plsc.subcore_barrier()
        pltpu.sync_copy(accum.at[pl.ds(s * RPS, RPS)],
                        part.at[c, pl.ds(s * RPS, RPS)])


def _sc_deg(dstp0, dstp1, dstp2):
    zz = jnp.zeros((RPS, 16), jnp.float32)
    call = pl.kernel(
        _deg_body,
        out_type=[jax.ShapeDtypeStruct((NC, NPAD, 16), jnp.float32)] * 3,
        mesh=plsc.VectorSubcoreMesh(core_axis_name="c", subcore_axis_name="s"),
        scratch_types=[
            pltpu.VMEM((KW, CHUNK), jnp.int32),
            pltpu.VMEM((CHUNK, 16), jnp.float32),
            pltpu.VMEM_SHARED((NPAD, 16), jnp.float32),
        ],
    )
    return call(dstp0, dstp1, dstp2, zz)


def _sc_edges(y0, srcp0, dstp0, y1, srcp1, dstp1, y2, srcp2, dstp2):
    zz = jnp.zeros((RPS, D), jnp.float32)
    call = pl.kernel(
        _edge_body,
        out_type=[jax.ShapeDtypeStruct((NC, NPAD, D), jnp.float32)] * 3,
        mesh=plsc.VectorSubcoreMesh(core_axis_name="c", subcore_axis_name="s"),
        scratch_types=[
            pltpu.VMEM((KW, CHUNK), jnp.int32),
            pltpu.VMEM((KW, CHUNK), jnp.int32),
            pltpu.VMEM((CHUNK, D), jnp.float32),
            pltpu.VMEM_SHARED((NPAD, D), jnp.float32),
            pltpu.SemaphoreType.DMA,
        ],
    )
    return call(y0, srcp0, dstp0, y1, srcp1, dstp1, y2, srcp2, dstp2, zz)


def _dinv_block(cref):
    cnt = cref[0][:, 0:1] + cref[1][:, 0:1] + 1.0
    return lax.rsqrt(cnt)


def _tc_proj_body(x0, w0, b0, c0, x1, w1, b1, c1, x2, w2, b2, c2,
                  y0, y1, y2):
    for x, wr, br, cr, yr in ((x0, w0, b0, c0, y0), (x1, w1, b1, c1, y1),
                              (x2, w2, b2, c2, y2)):
        xp = lax.dot_general(x[...], wr[...], (((1,), (1,)), ((), ())),
                             preferred_element_type=jnp.float32) + br[...]
        yr[...] = _dinv_block(cr) * xp


def _tc_out_body(p0, y0, c0, p1, y1, c1, p2, y2, c2, o0, o1, o2):
    for p, y, cr, o in ((p0, y0, c0, o0), (p1, y1, c1, o1),
                        (p2, y2, c2, o2)):
        o[...] = _dinv_block(cr) * (p[0] + p[1] + y[...])


def _tc_proj(xs, ws, bs, cnts):
    row = pl.BlockSpec((BR, D), lambda i: (i, 0))
    wspec = pl.BlockSpec((D, D), lambda i: (0, 0))
    bspec = pl.BlockSpec((1, D), lambda i: (0, 0))
    cspec = pl.BlockSpec((NC, BR, 16), lambda i: (0, i, 0))
    args = []
    specs = []
    for x, w, b, c in zip(xs, ws, bs, cnts):
        args += [x, w.reshape(D, D), b.reshape(1, D), c]
        specs += [row, wspec, bspec, cspec]
    return pl.pallas_call(
        _tc_proj_body,
        grid=(N // BR,),
        in_specs=specs,
        out_specs=[row] * 3,
        out_shape=[jax.ShapeDtypeStruct((N, D), jnp.float32)] * 3,
    )(*args)


def _tc_out(parts, ys, cnts):
    row = pl.BlockSpec((BR, D), lambda i: (i, 0))
    pspec = pl.BlockSpec((NC, BR, D), lambda i: (0, i, 0))
    cspec = pl.BlockSpec((NC, BR, 16), lambda i: (0, i, 0))
    args = []
    specs = []
    for p, y, c in zip(parts, ys, cnts):
        args += [p, y, c]
        specs += [pspec, row, cspec]
    return pl.pallas_call(
        _tc_out_body,
        grid=(N // BR,),
        in_specs=specs,
        out_specs=[row] * 3,
        out_shape=[jax.ShapeDtypeStruct((N, D), jnp.float32)] * 3,
    )(*args)


def _prep_edges(ei):
    src = ei[0].astype(jnp.int32).reshape(NW, EPW)
    dst = ei[1].astype(jnp.int32).reshape(NW, EPW)
    pad = EPW_PAD - EPW
    srcp = jnp.pad(src, ((0, 0), (0, pad))).reshape(NW, KW, CHUNK)
    dstp = jnp.pad(dst, ((0, 0), (0, pad)),
                   constant_values=DUMMY).reshape(NW, KW, CHUNK)
    return srcp, dstp


def kernel(X_HypGNet, X_CGNet, X_SGNet, edge_index_hyp, edge_index_cg,
           edge_index_sg, W_hyp, b_hyp, W_cg, b_cg, W_sg, b_sg):
    srcp0, dstp0 = _prep_edges(edge_index_hyp)
    srcp1, dstp1 = _prep_edges(edge_index_cg)
    srcp2, dstp2 = _prep_edges(edge_index_sg)

    cnts = _sc_deg(dstp0, dstp1, dstp2)

    ys = _tc_proj((X_HypGNet, X_CGNet, X_SGNet), (W_hyp, W_cg, W_sg),
                  (b_hyp, b_cg, b_sg), cnts)

    parts = _sc_edges(ys[0], srcp0, dstp0, ys[1], srcp1, dstp1,
                      ys[2], srcp2, dstp2)

    outs = _tc_out(parts, ys, cnts)
    return (outs[0], outs[1], outs[2])
